# Initial kernel scaffold; baseline (speedup 1.0000x reference)
import jax
import jax.numpy as jnp
from jax.experimental import pallas as pl


def kernel(x, W_hash, W_q, b_q, W_v, b_v, W_o, b_o):
    raise NotImplementedError("write your pallas kernel here")


# speedup vs baseline: 38.3383x; 38.3383x over previous
import jax
import jax.numpy as jnp
from jax import lax
from jax.experimental import pallas as pl
from jax.experimental.pallas import tpu as pltpu

B = 2
S = 2048
D = 1024
H = 16
DH = D // H
BS = 64
NB = S // BS
BH = B * H
TS = 256
EPS = 1e-4
SCALE = float(D) ** 0.5
NEG_INF = float("-inf")


def _proj_kernel(x_ref, wqt_ref, bq_ref, wvt_ref, bv_ref, whe_ref, who_ref,
                 q_ref, v_ref, ang_ref):
    xt = x_ref[0]
    q_ref[0] = jnp.dot(xt, wqt_ref[...], preferred_element_type=jnp.float32) + bq_ref[...]
    v_ref[0] = jnp.dot(xt, wvt_ref[...], preferred_element_type=jnp.float32) + bv_ref[...]
    num = jnp.dot(xt, whe_ref[...], preferred_element_type=jnp.float32)
    den = jnp.dot(xt, who_ref[...], preferred_element_type=jnp.float32)
    a = num / (den + EPS)
    ang_ref[...] = a.T


def _rank_kernel(a_ref, r_ref):
    arow = a_ref[0]
    acol = jnp.transpose(arow)
    for t in range(S // TS):
        asl = arow[:, t * TS:(t + 1) * TS]
        lt = acol < asl
        eq = acol == asl
        ti = lax.broadcasted_iota(jnp.int32, (S, TS), 0)
        si = t * TS + lax.broadcasted_iota(jnp.int32, (S, TS), 1)
        less = jnp.logical_or(lt, jnp.logical_and(eq, ti < si))
        cf = less.astype(jnp.float32)
        r_ref[0, :, t * TS:(t + 1) * TS] = jnp.sum(cf, axis=0, keepdims=True)


def _attn_kernel(rank_ref, q_ref, v_ref, out_ref, ho0_ref, qh_ref, vh_ref):
    rrow = rank_ref[0]
    qs = q_ref[0, 0]
    vs = v_ref[0, 0]
    for rt in range(S // TS):
        ri = (rt * TS + lax.broadcasted_iota(jnp.int32, (TS, S), 0)
              ).astype(jnp.float32)
        P = (rrow == ri).astype(jnp.float32)
        qh_ref[rt * TS:(rt + 1) * TS, :] = jnp.dot(
            P, qs, preferred_element_type=jnp.float32)
        vh_ref[rt * TS:(rt + 1) * TS, :] = jnp.dot(
            P, vs, preferred_element_type=jnp.float32)
    ii = lax.broadcasted_iota(jnp.int32, (BS, 2 * BS), 0)
    jj = lax.broadcasted_iota(jnp.int32, (BS, 2 * BS), 1)
    selfmask = ii == jj
    for n in range(NB):
        nn = (n + 1) % NB
        qn = qh_ref[n * BS:(n + 1) * BS, :]
        kn = jnp.concatenate(
            [qn, qh_ref[nn * BS:(nn + 1) * BS, :]], axis=0)
        vn = jnp.concatenate(
            [vh_ref[n * BS:(n + 1) * BS, :],
             vh_ref[nn * BS:(nn + 1) * BS, :]], axis=0)
        sc = lax.dot_general(qn, kn, (((1,), (1,)), ((), ())),
                             preferred_element_type=jnp.float32) / SCALE
        sc = jnp.where(selfmask, NEG_INF, sc)
        m = jnp.max(sc, axis=1, keepdims=True)
        e = jnp.exp(sc - m)
        p = e / jnp.sum(e, axis=1, keepdims=True)
        ho = jnp.dot(p, vn, preferred_element_type=jnp.float32)
        out_ref[0, 0, n * BS:(n + 1) * BS, :] = ho
        ho0_ref[0, :, n * BS:(n + 1) * BS] = jnp.transpose(ho[:, 0:1])


def _out_kernel(x_ref, ho0_ref, rank_ref, wot_ref, bo_ref, o_ref, c_ref):
    xt = x_ref[0]
    s0 = pl.program_id(1) * TS
    riota = lax.broadcasted_iota(jnp.int32, (S, TS), 0).astype(jnp.float32)

    def body(hd, carry):
        hrow = ho0_ref[hd, :, :]
        rsl = rank_ref[hd, :, pl.ds(s0, TS)]
        cmpT = (riota == rsl).astype(jnp.float32)
        o0 = jnp.dot(hrow, cmpT,
                     preferred_element_type=jnp.float32)
        c_ref[pl.ds(hd, 1), :] = o0
        return carry

    lax.fori_loop(0, H, body, 0)
    CT = jnp.transpose(c_ref[...])
    hi = lax.broadcasted_iota(jnp.int32, (H, D), 0)
    ci = lax.broadcasted_iota(jnp.int32, (H, D), 1)
    E = (ci == hi * DH).astype(jnp.float32)
    rep = jnp.dot(CT, E, preferred_element_type=jnp.float32)
    li = lax.broadcasted_iota(jnp.int32, (TS, D), 1)
    xm = jnp.where(li % DH == 0, rep, xt)
    o_ref[0] = jnp.dot(xm, wot_ref[...], preferred_element_type=jnp.float32) + bo_ref[...]


def kernel(x, W_hash, W_q, b_q, W_v, b_v, W_o, b_o):
    wqt = W_q.T
    wvt = W_v.T
    wot = W_o.T
    whe = W_hash[0::2].T
    who = W_hash[1::2].T
    bq2 = b_q[None, :]
    bv2 = b_v[None, :]
    bo2 = b_o[None, :]

    q, v, ang = pl.pallas_call(
        _proj_kernel,
        grid=(B, S // TS),
        in_specs=[
            pl.BlockSpec((1, TS, D), lambda b, t: (b, t, 0)),
            pl.BlockSpec((D, D), lambda b, t: (0, 0)),
            pl.BlockSpec((1, D), lambda b, t: (0, 0)),
            pl.BlockSpec((D, D), lambda b, t: (0, 0)),
            pl.BlockSpec((1, D), lambda b, t: (0, 0)),
            pl.BlockSpec((D, H), lambda b, t: (0, 0)),
            pl.BlockSpec((D, H), lambda b, t: (0, 0)),
        ],
        out_specs=[
            pl.BlockSpec((1, TS, D), lambda b, t: (b, t, 0)),
            pl.BlockSpec((1, TS, D), lambda b, t: (b, t, 0)),
            pl.BlockSpec((H, TS), lambda b, t: (b, t)),
        ],
        out_shape=[
            jax.ShapeDtypeStruct((B, S, D), jnp.float32),
            jax.ShapeDtypeStruct((B, S, D), jnp.float32),
            jax.ShapeDtypeStruct((BH, S), jnp.float32),
        ],
    )(x, wqt, bq2, wvt, bv2, whe, who)

    q4 = q.reshape(B, S, H, DH).transpose(0, 2, 1, 3)
    v4 = v.reshape(B, S, H, DH).transpose(0, 2, 1, 3)
    a3 = ang.reshape(BH, 1, S)
    rank = pl.pallas_call(
        _rank_kernel,
        grid=(BH,),
        in_specs=[pl.BlockSpec((1, 1, S), lambda i: (i, 0, 0))],
        out_specs=pl.BlockSpec((1, 1, S), lambda i: (i, 0, 0)),
        out_shape=jax.ShapeDtypeStruct((BH, 1, S), jnp.float32),
    )(a3)

    xs4 = pl.pallas_call(
        _attn_kernel,
        grid=(BH,),
        in_specs=[
            pl.BlockSpec((1, 1, S), lambda i: (i, 0, 0)),
            pl.BlockSpec((1, 1, S, DH), lambda i: (i // H, i % H, 0, 0)),
            pl.BlockSpec((1, 1, S, DH), lambda i: (i // H, i % H, 0, 0)),
        ],
        out_specs=[
            pl.BlockSpec((1, 1, S, DH), lambda i: (i // H, i % H, 0, 0)),
            pl.BlockSpec((1, 1, S), lambda i: (i, 0, 0)),
        ],
        out_shape=[
            jax.ShapeDtypeStruct((B, H, S, DH), jnp.float32),
            jax.ShapeDtypeStruct((BH, 1, S), jnp.float32),
        ],
        scratch_shapes=[
            pltpu.VMEM((S, DH), jnp.float32),
            pltpu.VMEM((S, DH), jnp.float32),
        ],
    )(rank, q4, v4)
    xs4, ho0 = xs4
    xs = xs4.transpose(0, 2, 1, 3).reshape(B, S, D)

    out = pl.pallas_call(
        _out_kernel,
        grid=(B, S // TS),
        in_specs=[
            pl.BlockSpec((1, TS, D), lambda b, t: (b, t, 0)),
            pl.BlockSpec((H, 1, S), lambda b, t: (b, 0, 0)),
            pl.BlockSpec((H, 1, S), lambda b, t: (b, 0, 0)),
            pl.BlockSpec((D, D), lambda b, t: (0, 0)),
            pl.BlockSpec((1, D), lambda b, t: (0, 0)),
        ],
        out_specs=pl.BlockSpec((1, TS, D), lambda b, t: (b, t, 0)),
        out_shape=jax.ShapeDtypeStruct((B, S, D), jnp.float32),
        scratch_shapes=[pltpu.VMEM((H, TS), jnp.float32)],
    )(xs, ho0, rank, wot, bo2)

    return out


# speedup vs baseline: 46.1294x; 1.2032x over previous
import jax
import jax.numpy as jnp
from jax import lax
from jax.experimental import pallas as pl
from jax.experimental.pallas import tpu as pltpu
from jax.experimental.pallas import tpu_sc as plsc

B = 2
S = 2048
D = 1024
H = 16
DH = D // H
BS = 64
NB = S // BS
BH = B * H
TS = 256
EPS = 1e-4
SCALE = float(D) ** 0.5
NEG_INF = float("-inf")

L = 16
CH = 128
NCH = S // CH


def _proj_kernel(x_ref, wqt_ref, bq_ref, wvt_ref, bv_ref, whe_ref, who_ref,
                 q_ref, v_ref, ang_ref):
    xt = x_ref[0]
    q_ref[0] = jnp.dot(xt, wqt_ref[...], preferred_element_type=jnp.float32) + bq_ref[...]
    v_ref[0] = jnp.dot(xt, wvt_ref[...], preferred_element_type=jnp.float32) + bv_ref[...]
    num = jnp.dot(xt, whe_ref[...], preferred_element_type=jnp.float32)
    den = jnp.dot(xt, who_ref[...], preferred_element_type=jnp.float32)
    a = num / (den + EPS)
    ang_ref[...] = a.T


def _rank_kernel(a_ref, r_ref):
    arow = a_ref[0]
    acol = jnp.transpose(arow)
    for t in range(S // TS):
        asl = arow[:, t * TS:(t + 1) * TS]
        lt = acol < asl
        eq = acol == asl
        ti = lax.broadcasted_iota(jnp.int32, (S, TS), 0)
        si = t * TS + lax.broadcasted_iota(jnp.int32, (S, TS), 1)
        less = jnp.logical_or(lt, jnp.logical_and(eq, ti < si))
        cf = less.astype(jnp.float32)
        r_ref[0, :, t * TS:(t + 1) * TS] = jnp.sum(
            cf, axis=0, keepdims=True).astype(jnp.int32)


def _sc_sort_kernel(rank_hbm, qv_hbm, qvs_hbm, rank_v, perm_v, qvbuf, sem):
    wid = lax.axis_index("s") * 2 + lax.axis_index("c")
    base = wid * S
    pltpu.sync_copy(rank_hbm.at[pl.ds(base, S)], rank_v)

    def perm_body(i, carry):
        r16 = rank_v[pl.ds(i * L, L)]
        vals = base + i * L + lax.iota(jnp.int32, L)
        plsc.store_scatter(perm_v, [r16], vals)
        return carry

    lax.fori_loop(0, S // L, perm_body, 0)

    def chunk_body(c, carry):
        off = c * CH
        idx = perm_v.at[pl.ds(off, CH)]
        pltpu.async_copy(qv_hbm.at[idx], qvbuf, sem).wait()
        pltpu.sync_copy(qvbuf, qvs_hbm.at[pl.ds(base + off, CH)])
        return carry

    lax.fori_loop(0, NCH, chunk_body, 0)


def _attn_kernel(qv_ref, out_ref, ho0_ref):
    ii = lax.broadcasted_iota(jnp.int32, (BS, 2 * BS), 0)
    jj = lax.broadcasted_iota(jnp.int32, (BS, 2 * BS), 1)
    selfmask = ii == jj
    for n in range(NB):
        nn = (n + 1) % NB
        qn = qv_ref[0, n * BS:(n + 1) * BS, 0:DH]
        kn = jnp.concatenate(
            [qn, qv_ref[0, nn * BS:(nn + 1) * BS, 0:DH]], axis=0)
        vn = jnp.concatenate(
            [qv_ref[0, n * BS:(n + 1) * BS, DH:2 * DH],
             qv_ref[0, nn * BS:(nn + 1) * BS, DH:2 * DH]], axis=0)
        sc = lax.dot_general(qn, kn, (((1,), (1,)), ((), ())),
                             preferred_element_type=jnp.float32) / SCALE
        sc = jnp.where(selfmask, NEG_INF, sc)
        m = jnp.max(sc, axis=1, keepdims=True)
        e = jnp.exp(sc - m)
        p = e / jnp.sum(e, axis=1, keepdims=True)
        ho = jnp.dot(p, vn, preferred_element_type=jnp.float32)
        out_ref[0, n * BS:(n + 1) * BS, :] = ho
        ho0_ref[0, :, n * BS:(n + 1) * BS] = jnp.transpose(ho[:, 0:1])


def _sc_unsort_kernel(rank_hbm, ho0_hbm, c0_hbm, rank_v, ho0_v, c0_v):
    wid = lax.axis_index("s") * 2 + lax.axis_index("c")
    base = wid * S
    pltpu.sync_copy(rank_hbm.at[pl.ds(base, S)], rank_v)
    pltpu.sync_copy(ho0_hbm.at[pl.ds(base, S)], ho0_v)

    def body(i, carry):
        r16 = rank_v[pl.ds(i * L, L)]
        c0_v[pl.ds(i * L, L)] = plsc.load_gather(ho0_v, [r16])
        return carry

    lax.fori_loop(0, S // L, body, 0)
    pltpu.sync_copy(c0_v, c0_hbm.at[pl.ds(base, S)])


def _out_kernel(x_ref, c0_ref, wot_ref, bo_ref, o_ref):
    xt = x_ref[0]
    CT = jnp.transpose(c0_ref[...])
    hi = lax.broadcasted_iota(jnp.int32, (H, D), 0)
    ci = lax.broadcasted_iota(jnp.int32, (H, D), 1)
    E = (ci == hi * DH).astype(jnp.float32)
    rep = jnp.dot(CT, E, preferred_element_type=jnp.float32)
    li = lax.broadcasted_iota(jnp.int32, (TS, D), 1)
    xm = jnp.where(li % DH == 0, rep, xt)
    o_ref[0] = jnp.dot(xm, wot_ref[...], preferred_element_type=jnp.float32) + bo_ref[...]


_SC_MESH = plsc.VectorSubcoreMesh(core_axis_name="c", subcore_axis_name="s")


def kernel(x, W_hash, W_q, b_q, W_v, b_v, W_o, b_o):
    wqt = W_q.T
    wvt = W_v.T
    wot = W_o.T
    whe = W_hash[0::2].T
    who = W_hash[1::2].T
    bq2 = b_q[None, :]
    bv2 = b_v[None, :]
    bo2 = b_o[None, :]

    q, v, ang = pl.pallas_call(
        _proj_kernel,
        grid=(B, S // TS),
        in_specs=[
            pl.BlockSpec((1, TS, D), lambda b, t: (b, t, 0)),
            pl.BlockSpec((D, D), lambda b, t: (0, 0)),
            pl.BlockSpec((1, D), lambda b, t: (0, 0)),
            pl.BlockSpec((D, D), lambda b, t: (0, 0)),
            pl.BlockSpec((1, D), lambda b, t: (0, 0)),
            pl.BlockSpec((D, H), lambda b, t: (0, 0)),
            pl.BlockSpec((D, H), lambda b, t: (0, 0)),
        ],
        out_specs=[
            pl.BlockSpec((1, TS, D), lambda b, t: (b, t, 0)),
            pl.BlockSpec((1, TS, D), lambda b, t: (b, t, 0)),
            pl.BlockSpec((H, TS), lambda b, t: (b, t)),
        ],
        out_shape=[
            jax.ShapeDtypeStruct((B, S, D), jnp.float32),
            jax.ShapeDtypeStruct((B, S, D), jnp.float32),
            jax.ShapeDtypeStruct((BH, S), jnp.float32),
        ],
    )(x, wqt, bq2, wvt, bv2, whe, who)

    a3 = ang.reshape(BH, 1, S)
    rank = pl.pallas_call(
        _rank_kernel,
        grid=(BH,),
        in_specs=[pl.BlockSpec((1, 1, S), lambda i: (i, 0, 0))],
        out_specs=pl.BlockSpec((1, 1, S), lambda i: (i, 0, 0)),
        out_shape=jax.ShapeDtypeStruct((BH, 1, S), jnp.int32),
    )(a3)
    rank_flat = rank.reshape(BH * S)

    q4 = q.reshape(B, S, H, DH).transpose(0, 2, 1, 3)
    v4 = v.reshape(B, S, H, DH).transpose(0, 2, 1, 3)
    qv4f = jnp.concatenate([q4, v4], axis=3).reshape(BH * S, 2 * DH)

    qvs = pl.kernel(
        _sc_sort_kernel,
        compiler_params=pltpu.CompilerParams(needs_layout_passes=False),
        out_type=jax.ShapeDtypeStruct((BH * S, 2 * DH), jnp.float32),
        mesh=_SC_MESH,
        scratch_types=[
            pltpu.VMEM((S,), jnp.int32),
            pltpu.VMEM((S,), jnp.int32),
            pltpu.VMEM((CH, 2 * DH), jnp.float32),
            pltpu.SemaphoreType.DMA,
        ],
    )(rank_flat, qv4f)

    xs3, ho0 = pl.pallas_call(
        _attn_kernel,
        grid=(BH,),
        in_specs=[
            pl.BlockSpec((1, S, 2 * DH), lambda i: (i, 0, 0)),
        ],
        out_specs=[
            pl.BlockSpec((1, S, DH), lambda i: (i, 0, 0)),
            pl.BlockSpec((1, 1, S), lambda i: (i, 0, 0)),
        ],
        out_shape=[
            jax.ShapeDtypeStruct((BH, S, DH), jnp.float32),
            jax.ShapeDtypeStruct((BH, 1, S), jnp.float32),
        ],
    )(qvs.reshape(BH, S, 2 * DH))
    xs = xs3.reshape(B, H, S, DH).transpose(0, 2, 1, 3).reshape(B, S, D)

    c0 = pl.kernel(
        _sc_unsort_kernel,
        compiler_params=pltpu.CompilerParams(needs_layout_passes=False),
        out_type=jax.ShapeDtypeStruct((BH * S,), jnp.float32),
        mesh=_SC_MESH,
        scratch_types=[
            pltpu.VMEM((S,), jnp.int32),
            pltpu.VMEM((S,), jnp.float32),
            pltpu.VMEM((S,), jnp.float32),
        ],
    )(rank_flat, ho0.reshape(BH * S))

    out = pl.pallas_call(
        _out_kernel,
        grid=(B, S // TS),
        in_specs=[
            pl.BlockSpec((1, TS, D), lambda b, t: (b, t, 0)),
            pl.BlockSpec((H, TS), lambda b, t: (b, t)),
            pl.BlockSpec((D, D), lambda b, t: (0, 0)),
            pl.BlockSpec((1, D), lambda b, t: (0, 0)),
        ],
        out_specs=pl.BlockSpec((1, TS, D), lambda b, t: (b, t, 0)),
        out_shape=jax.ShapeDtypeStruct((B, S, D), jnp.float32),
    )(xs, c0.reshape(BH, S), wot, bo2)

    return out
